# Initial kernel scaffold; baseline (speedup 1.0000x reference)
import jax
import jax.numpy as jnp
from jax.experimental import pallas as pl


def kernel(input_, weight):
    raise NotImplementedError("write your pallas kernel here")


# speedup vs baseline: 1.8297x; 1.8297x over previous
import functools

import jax
import jax.numpy as jnp
from jax import lax
from jax.experimental import pallas as pl
from jax.experimental.pallas import tpu as pltpu
from jax.experimental.pallas import tpu_sc as plsc

NUM_EMB = 1_000_000
DIM = 64
B_FLAT = 16384 * 50
IDX_MINOR = 128
IDX_ROWS = B_FLAT // IDX_MINOR
NUM_WORKERS = 32
IDX_ROWS_PER_W = IDX_ROWS // NUM_WORKERS
GATHERS_PER_STEP = 4
STEP_ROWS = GATHERS_PER_STEP * IDX_MINOR
STEPS = IDX_ROWS_PER_W // GATHERS_PER_STEP
ROWS_PER_W = IDX_ROWS_PER_W * IDX_MINOR


def _make_sc_gather():
    mesh = plsc.VectorSubcoreMesh(core_axis_name="c", subcore_axis_name="s")

    @functools.partial(
        pl.kernel,
        out_type=jax.ShapeDtypeStruct((B_FLAT, DIM), jnp.float32),
        mesh=mesh,
        scratch_types=[
            pltpu.VMEM((IDX_ROWS_PER_W, IDX_MINOR), jnp.int32),
            pltpu.VMEM((STEP_ROWS, DIM), jnp.float32),
            pltpu.SemaphoreType.DMA,
        ],
        compiler_params=pltpu.CompilerParams(use_tc_tiling_on_sc=False),
    )
    def emb(idx_hbm, table_hbm, out_hbm, idx_v, rows_v, sem):
        wid = lax.axis_index("s") * 2 + lax.axis_index("c")
        pltpu.sync_copy(idx_hbm.at[pl.ds(wid * IDX_ROWS_PER_W, IDX_ROWS_PER_W)],
                        idx_v)
        out_base = wid * ROWS_PER_W

        def step(s, carry):
            copies = []
            for j in range(GATHERS_PER_STEP):
                copies.append(pltpu.async_copy(
                    table_hbm.at[idx_v.at[s * GATHERS_PER_STEP + j]],
                    rows_v.at[pl.ds(j * IDX_MINOR, IDX_MINOR)],
                    sem))
            for c in copies:
                c.wait()
            pltpu.sync_copy(rows_v,
                            out_hbm.at[pl.ds(out_base + s * STEP_ROWS,
                                             STEP_ROWS)])
            return carry

        lax.fori_loop(0, STEPS, step, 0)

    return emb


_sc_gather = _make_sc_gather()


def kernel(input_, weight):
    idx = input_.astype(jnp.int32).reshape(IDX_ROWS, IDX_MINOR)
    out = _sc_gather(idx, weight)
    return out.reshape(input_.shape[0], input_.shape[1], DIM)


# speedup vs baseline: 1.8700x; 1.0220x over previous
import functools

import jax
import jax.numpy as jnp
from jax import lax
from jax.experimental import pallas as pl
from jax.experimental.pallas import tpu as pltpu
from jax.experimental.pallas import tpu_sc as plsc

NUM_EMB = 1_000_000
DIM = 64
B_FLAT = 16384 * 50
IDX_MINOR = 128
IDX_ROWS = B_FLAT // IDX_MINOR
NUM_WORKERS = 32
IDX_ROWS_PER_W = IDX_ROWS // NUM_WORKERS
GATHERS_PER_STEP = 5
STEP_ROWS = GATHERS_PER_STEP * IDX_MINOR
STEPS = IDX_ROWS_PER_W // GATHERS_PER_STEP
ROWS_PER_W = IDX_ROWS_PER_W * IDX_MINOR
STEP_BYTES_SHAPE = jax.ShapeDtypeStruct((STEP_ROWS, DIM), jnp.float32)


def _make_sc_gather():
    mesh = plsc.VectorSubcoreMesh(core_axis_name="c", subcore_axis_name="s")

    @functools.partial(
        pl.kernel,
        out_type=jax.ShapeDtypeStruct((B_FLAT, DIM), jnp.float32),
        mesh=mesh,
        scratch_types=[
            pltpu.VMEM((IDX_ROWS_PER_W, IDX_MINOR), jnp.int32),
            pltpu.VMEM((STEP_ROWS, DIM), jnp.float32),
            pltpu.VMEM((STEP_ROWS, DIM), jnp.float32),
            pltpu.SemaphoreType.DMA,
            pltpu.SemaphoreType.DMA,
        ],
        compiler_params=pltpu.CompilerParams(use_tc_tiling_on_sc=False),
    )
    def emb(idx_hbm, table_hbm, out_hbm, idx_v, rows_a, rows_b, sem_g, sem_o):
        wid = lax.axis_index("s") * 2 + lax.axis_index("c")
        pltpu.sync_copy(idx_hbm.at[pl.ds(wid * IDX_ROWS_PER_W, IDX_ROWS_PER_W)],
                        idx_v)
        out_base = wid * ROWS_PER_W
        bufs = (rows_a, rows_b)

        def fire_gathers(s, buf):
            for j in range(GATHERS_PER_STEP):
                pltpu.async_copy(
                    table_hbm.at[idx_v.at[s * GATHERS_PER_STEP + j]],
                    buf.at[pl.ds(j * IDX_MINOR, IDX_MINOR)],
                    sem_g)

        def drain(sem, buf):
            pltpu.make_async_copy(out_hbm.at[pl.ds(0, STEP_ROWS)], buf,
                                  sem).wait()

        fire_gathers(0, bufs[0])

        def macro(m, carry):
            for b in range(2):
                s = m * 2 + b
                drain(sem_g, bufs[b])

                @pl.when(s > 0)
                def _():
                    drain(sem_o, bufs[1 - b])

                @pl.when(s < STEPS - 1)
                def _():
                    fire_gathers(s + 1, bufs[1 - b])

                pltpu.async_copy(
                    bufs[b],
                    out_hbm.at[pl.ds(out_base + s * STEP_ROWS, STEP_ROWS)],
                    sem_o)
            return carry

        lax.fori_loop(0, STEPS // 2, macro, 0)
        drain(sem_o, bufs[1])

    return emb


_sc_gather = _make_sc_gather()


def kernel(input_, weight):
    idx = input_.astype(jnp.int32).reshape(IDX_ROWS, IDX_MINOR)
    out = _sc_gather(idx, weight)
    return out.reshape(input_.shape[0], input_.shape[1], DIM)
